# baseline (speedup 1.0000x reference)
import dataclasses
import functools
import math

import jax
import jax.numpy as jnp
from jax import lax
from jax.experimental import pallas as pl
from jax.experimental.pallas import tpu as pltpu
from jax.experimental.pallas import tpu_sc as plsc

_N = 32000
_B = 4096
_PAD = 0
_SMOOTH = 0.1
_CONF = 1.0 - _SMOOTH
_EPS = _SMOOTH / (_N - 2)
_C1 = _CONF * math.log(_CONF) + _SMOOTH * math.log(_EPS)
_CME = _CONF - _EPS

_WSC = 0
_NTC = _N - _WSC

_BR = 1024
_K = 5
_CB = 5
_W = _NTC // (_K * _CB)

_NW = 32
_PW = _B // _NW
_SCL = 16


def _sc_gather(x, t32):
    mesh = plsc.VectorSubcoreMesh(core_axis_name="c", subcore_axis_name="s")
    cp = pltpu.CompilerParams()
    if "needs_layout_passes" in pltpu.CompilerParams.__dataclass_fields__:
        cp = dataclasses.replace(cp, needs_layout_passes=False)

    @functools.partial(
        pl.kernel,
        out_type=jax.ShapeDtypeStruct((_B,), jnp.float32),
        mesh=mesh,
        compiler_params=cp,
        scratch_types=[
            pltpu.VMEM((_PW,), jnp.int32),
            pltpu.VMEM((_SCL, 8, 128), jnp.float32),
            pltpu.VMEM((_PW,), jnp.float32),
            pltpu.SemaphoreType.DMA,
        ],
    )
    def gather_kernel(x_hbm, t_hbm, out_hbm, t_v, vals_v, out_v, dsem):
        wid = lax.axis_index("s") * 2 + lax.axis_index("c")
        base = wid * _PW
        pltpu.sync_copy(t_hbm.at[pl.ds(base, _PW)], t_v)

        for b in range(_PW // _SCL):
            tvb = t_v[pl.ds(b * _SCL, _SCL)]
            copies = []
            for u in range(_SCL):
                i = b * _SCL + u
                row0 = base + (i // 8) * 8
                c0 = (tvb[u] // 128) * 128
                copies.append(pltpu.async_copy(
                    x_hbm.at[pl.ds(row0, 8), pl.ds(c0, 128)],
                    vals_v.at[u], dsem))
            for cp in copies:
                cp.wait()
            bidx = lax.iota(jnp.int32, _SCL)
            ridx = bidx % 8
            cidx = tvb % 128
            g16 = plsc.load_gather(vals_v, [bidx, ridx, cidx])
            out_v[pl.ds(b * _SCL, _SCL)] = g16

        pltpu.sync_copy(out_v, out_hbm.at[pl.ds(base, _PW)])

    return gather_kernel(x, t32)


def _sc_colsum(x):
    mesh = plsc.VectorSubcoreMesh(core_axis_name="c", subcore_axis_name="s")

    @functools.partial(
        pl.kernel,
        out_type=jax.ShapeDtypeStruct((_B, _SCL), jnp.float32),
        mesh=mesh,
    )
    def colsum_kernel(x_hbm, out_hbm):
        def body(in_vmem, out_vmem):
            for j in range(8):
                def chunk(ci, acc):
                    base = ci * 128
                    for k in range(128 // _SCL):
                        acc = acc + in_vmem[j, pl.ds(base + k * _SCL, _SCL)]
                    return acc
                acc = lax.fori_loop(
                    0, _WSC // 128, chunk,
                    jnp.zeros((_SCL,), jnp.float32))
                out_vmem[j, :] = acc

        pltpu.emit_pipeline(
            body,
            grid=(_B // 8,),
            in_specs=[pl.BlockSpec((8, _WSC), lambda i: (i, _NTC // _WSC))],
            out_specs=[pl.BlockSpec((8, _SCL), lambda i: (i, 0))],
            core_axis_name=("c", "s"),
            dimension_semantics=(pltpu.PARALLEL,),
        )(x_hbm, out_hbm)

    return colsum_kernel(x)


def _tc_rowsum_body(*refs):
    x_refs = refs[:_K]
    out_ref, acc_ref = refs[_K:]
    c = pl.program_id(1)
    nc = pl.num_programs(1)

    part = None
    for k in range(_K):
        xb = x_refs[k][...]
        for j in range(_W // 128):
            s = xb[:, j * 128:(j + 1) * 128]
            if k == 0 and j == 0:
                colid = lax.broadcasted_iota(jnp.int32, (_BR, 128), 1)
                s = jnp.where((colid == 0) & (c == 0), 0.0, s)
            part = s if part is None else part + s

    @pl.when(c == 0)
    def _():
        acc_ref[...] = part

    @pl.when(c != 0)
    def _():
        acc_ref[...] = acc_ref[...] + part

    @pl.when(c == nc - 1)
    def _():
        out_ref[...] = acc_ref[...].sum(axis=1)


def _tc_rowsum(x):
    grid = (_B // _BR, _CB)
    stripe_specs = [
        pl.BlockSpec((_BR, _W), functools.partial(
            lambda k, r, c: (r, k * _CB + c), k))
        for k in range(_K)
    ]
    return pl.pallas_call(
        _tc_rowsum_body,
        grid=grid,
        in_specs=stripe_specs,
        out_specs=pl.BlockSpec((_BR,), lambda r, c: (r,)),
        out_shape=jax.ShapeDtypeStruct((_B,), jnp.float32),
        scratch_shapes=[pltpu.VMEM((_BR, 128), jnp.float32)],
    )(*([x] * _K))


def _tc_combine_body(rs_ref, g_ref, t_ref, out_ref):
    rowsum = rs_ref[...]
    m = t_ref[...] != _PAD
    row_loss = jnp.where(m, _C1 - _CME * g_ref[...] - _EPS * rowsum, 0.0)
    out_ref[0, 0] = jnp.sum(row_loss)


def _tc_combine(rs_tc, g, t32):
    return pl.pallas_call(
        _tc_combine_body,
        out_specs=pl.BlockSpec(memory_space=pltpu.SMEM),
        out_shape=jax.ShapeDtypeStruct((1, 1), jnp.float32),
    )(rs_tc, g, t32)


def kernel(x, target):
    t32 = target.astype(jnp.int32)
    g = _sc_gather(x, t32)
    rs_tc = _tc_rowsum(x)
    loss = _tc_combine(rs_tc, g, t32)
    return loss[0, 0]


# speedup vs baseline: 1.0195x; 1.0195x over previous
import dataclasses
import functools
import math

import jax
import jax.numpy as jnp
from jax import lax
from jax.experimental import pallas as pl
from jax.experimental.pallas import tpu as pltpu
from jax.experimental.pallas import tpu_sc as plsc

_N = 32000
_B = 4096
_PAD = 0
_SMOOTH = 0.1
_CONF = 1.0 - _SMOOTH
_EPS = _SMOOTH / (_N - 2)
_C1 = _CONF * math.log(_CONF) + _SMOOTH * math.log(_EPS)
_CME = _CONF - _EPS

_WSC = 0
_NTC = _N - _WSC

_BR = 256
_K = 5
_CB = 5
_W = _NTC // (_K * _CB)

_NW = 32
_PW = _B // _NW
_SCL = 16


def _sc_gather(x, t32):
    mesh = plsc.VectorSubcoreMesh(core_axis_name="c", subcore_axis_name="s")
    cp = pltpu.CompilerParams()
    if "needs_layout_passes" in pltpu.CompilerParams.__dataclass_fields__:
        cp = dataclasses.replace(cp, needs_layout_passes=False)

    @functools.partial(
        pl.kernel,
        out_type=jax.ShapeDtypeStruct((_B,), jnp.float32),
        mesh=mesh,
        compiler_params=cp,
        scratch_types=[
            pltpu.VMEM((_PW,), jnp.int32),
            pltpu.VMEM((_SCL, 8, 128), jnp.float32),
            pltpu.VMEM((_PW,), jnp.float32),
            pltpu.SemaphoreType.DMA,
        ],
    )
    def gather_kernel(x_hbm, t_hbm, out_hbm, t_v, vals_v, out_v, dsem):
        wid = lax.axis_index("s") * 2 + lax.axis_index("c")
        base = wid * _PW
        pltpu.sync_copy(t_hbm.at[pl.ds(base, _PW)], t_v)

        for b in range(_PW // _SCL):
            tvb = t_v[pl.ds(b * _SCL, _SCL)]
            copies = []
            for u in range(_SCL):
                i = b * _SCL + u
                row0 = base + (i // 8) * 8
                c0 = (tvb[u] // 128) * 128
                copies.append(pltpu.async_copy(
                    x_hbm.at[pl.ds(row0, 8), pl.ds(c0, 128)],
                    vals_v.at[u], dsem))
            for cp in copies:
                cp.wait()
            bidx = lax.iota(jnp.int32, _SCL)
            ridx = bidx % 8
            cidx = tvb % 128
            g16 = plsc.load_gather(vals_v, [bidx, ridx, cidx])
            out_v[pl.ds(b * _SCL, _SCL)] = g16

        pltpu.sync_copy(out_v, out_hbm.at[pl.ds(base, _PW)])

    return gather_kernel(x, t32)


def _sc_colsum(x):
    mesh = plsc.VectorSubcoreMesh(core_axis_name="c", subcore_axis_name="s")

    @functools.partial(
        pl.kernel,
        out_type=jax.ShapeDtypeStruct((_B, _SCL), jnp.float32),
        mesh=mesh,
    )
    def colsum_kernel(x_hbm, out_hbm):
        def body(in_vmem, out_vmem):
            for j in range(8):
                def chunk(ci, acc):
                    base = ci * 128
                    for k in range(128 // _SCL):
                        acc = acc + in_vmem[j, pl.ds(base + k * _SCL, _SCL)]
                    return acc
                acc = lax.fori_loop(
                    0, _WSC // 128, chunk,
                    jnp.zeros((_SCL,), jnp.float32))
                out_vmem[j, :] = acc

        pltpu.emit_pipeline(
            body,
            grid=(_B // 8,),
            in_specs=[pl.BlockSpec((8, _WSC), lambda i: (i, _NTC // _WSC))],
            out_specs=[pl.BlockSpec((8, _SCL), lambda i: (i, 0))],
            core_axis_name=("c", "s"),
            dimension_semantics=(pltpu.PARALLEL,),
        )(x_hbm, out_hbm)

    return colsum_kernel(x)


def _tc_rowsum_body(*refs):
    x_refs = refs[:_K]
    out_ref, acc_ref = refs[_K:]
    c = pl.program_id(1)
    nc = pl.num_programs(1)

    part = None
    for k in range(_K):
        xb = x_refs[k][...]
        for j in range(_W // 128):
            s = xb[:, j * 128:(j + 1) * 128]
            if k == 0 and j == 0:
                colid = lax.broadcasted_iota(jnp.int32, (_BR, 128), 1)
                s = jnp.where((colid == 0) & (c == 0), 0.0, s)
            part = s if part is None else part + s

    @pl.when(c == 0)
    def _():
        acc_ref[...] = part

    @pl.when(c != 0)
    def _():
        acc_ref[...] = acc_ref[...] + part

    @pl.when(c == nc - 1)
    def _():
        out_ref[...] = acc_ref[...].sum(axis=1)


def _tc_rowsum(x):
    grid = (_B // _BR, _CB)
    stripe_specs = [
        pl.BlockSpec((_BR, _W), functools.partial(
            lambda k, r, c: (r, k * _CB + c), k))
        for k in range(_K)
    ]
    return pl.pallas_call(
        _tc_rowsum_body,
        grid=grid,
        in_specs=stripe_specs,
        out_specs=pl.BlockSpec((_BR,), lambda r, c: (r,)),
        out_shape=jax.ShapeDtypeStruct((_B,), jnp.float32),
        scratch_shapes=[pltpu.VMEM((_BR, 128), jnp.float32)],
    )(*([x] * _K))


def _tc_combine_body(rs_ref, g_ref, t_ref, out_ref):
    rowsum = rs_ref[...]
    m = t_ref[...] != _PAD
    row_loss = jnp.where(m, _C1 - _CME * g_ref[...] - _EPS * rowsum, 0.0)
    out_ref[0, 0] = jnp.sum(row_loss)


def _tc_combine(rs_tc, g, t32):
    return pl.pallas_call(
        _tc_combine_body,
        out_specs=pl.BlockSpec(memory_space=pltpu.SMEM),
        out_shape=jax.ShapeDtypeStruct((1, 1), jnp.float32),
    )(rs_tc, g, t32)


def kernel(x, target):
    t32 = target.astype(jnp.int32)
    g = _sc_gather(x, t32)
    rs_tc = _tc_rowsum(x)
    loss = _tc_combine(rs_tc, g, t32)
    return loss[0, 0]


# speedup vs baseline: 1.0284x; 1.0087x over previous
import dataclasses
import functools
import math

import jax
import jax.numpy as jnp
from jax import lax
from jax.experimental import pallas as pl
from jax.experimental.pallas import tpu as pltpu
from jax.experimental.pallas import tpu_sc as plsc

_N = 32000
_B = 4096
_PAD = 0
_SMOOTH = 0.1
_CONF = 1.0 - _SMOOTH
_EPS = _SMOOTH / (_N - 2)
_C1 = _CONF * math.log(_CONF) + _SMOOTH * math.log(_EPS)
_CME = _CONF - _EPS

_WSC = 0
_NTC = _N - _WSC

_BR = 128
_K = 1
_CB = 1
_W = _NTC // (_K * _CB)

_NW = 32
_PW = _B // _NW
_SCL = 16


def _sc_gather(x, t32):
    mesh = plsc.VectorSubcoreMesh(core_axis_name="c", subcore_axis_name="s")
    cp = pltpu.CompilerParams()
    if "needs_layout_passes" in pltpu.CompilerParams.__dataclass_fields__:
        cp = dataclasses.replace(cp, needs_layout_passes=False)

    @functools.partial(
        pl.kernel,
        out_type=jax.ShapeDtypeStruct((_B,), jnp.float32),
        mesh=mesh,
        compiler_params=cp,
        scratch_types=[
            pltpu.VMEM((_PW,), jnp.int32),
            pltpu.VMEM((_SCL, 8, 128), jnp.float32),
            pltpu.VMEM((_PW,), jnp.float32),
            pltpu.SemaphoreType.DMA,
        ],
    )
    def gather_kernel(x_hbm, t_hbm, out_hbm, t_v, vals_v, out_v, dsem):
        wid = lax.axis_index("s") * 2 + lax.axis_index("c")
        base = wid * _PW
        pltpu.sync_copy(t_hbm.at[pl.ds(base, _PW)], t_v)

        for b in range(_PW // _SCL):
            tvb = t_v[pl.ds(b * _SCL, _SCL)]
            copies = []
            for u in range(_SCL):
                i = b * _SCL + u
                row0 = base + (i // 8) * 8
                c0 = (tvb[u] // 128) * 128
                copies.append(pltpu.async_copy(
                    x_hbm.at[pl.ds(row0, 8), pl.ds(c0, 128)],
                    vals_v.at[u], dsem))
            for cp in copies:
                cp.wait()
            bidx = lax.iota(jnp.int32, _SCL)
            ridx = bidx % 8
            cidx = tvb % 128
            g16 = plsc.load_gather(vals_v, [bidx, ridx, cidx])
            out_v[pl.ds(b * _SCL, _SCL)] = g16

        pltpu.sync_copy(out_v, out_hbm.at[pl.ds(base, _PW)])

    return gather_kernel(x, t32)


def _sc_colsum(x):
    mesh = plsc.VectorSubcoreMesh(core_axis_name="c", subcore_axis_name="s")

    @functools.partial(
        pl.kernel,
        out_type=jax.ShapeDtypeStruct((_B, _SCL), jnp.float32),
        mesh=mesh,
    )
    def colsum_kernel(x_hbm, out_hbm):
        def body(in_vmem, out_vmem):
            for j in range(8):
                def chunk(ci, acc):
                    base = ci * 128
                    for k in range(128 // _SCL):
                        acc = acc + in_vmem[j, pl.ds(base + k * _SCL, _SCL)]
                    return acc
                acc = lax.fori_loop(
                    0, _WSC // 128, chunk,
                    jnp.zeros((_SCL,), jnp.float32))
                out_vmem[j, :] = acc

        pltpu.emit_pipeline(
            body,
            grid=(_B // 8,),
            in_specs=[pl.BlockSpec((8, _WSC), lambda i: (i, _NTC // _WSC))],
            out_specs=[pl.BlockSpec((8, _SCL), lambda i: (i, 0))],
            core_axis_name=("c", "s"),
            dimension_semantics=(pltpu.PARALLEL,),
        )(x_hbm, out_hbm)

    return colsum_kernel(x)


def _tc_rowsum_body(*refs):
    x_refs = refs[:_K]
    out_ref, acc_ref = refs[_K:]
    c = pl.program_id(1)
    nc = pl.num_programs(1)

    part = None
    for k in range(_K):
        xb = x_refs[k][...]
        for j in range(_W // 128):
            s = xb[:, j * 128:(j + 1) * 128]
            if k == 0 and j == 0:
                colid = lax.broadcasted_iota(jnp.int32, (_BR, 128), 1)
                s = jnp.where((colid == 0) & (c == 0), 0.0, s)
            part = s if part is None else part + s

    @pl.when(c == 0)
    def _():
        acc_ref[...] = part

    @pl.when(c != 0)
    def _():
        acc_ref[...] = acc_ref[...] + part

    @pl.when(c == nc - 1)
    def _():
        out_ref[...] = acc_ref[...].sum(axis=1)


def _tc_rowsum(x):
    grid = (_B // _BR, _CB)
    stripe_specs = [
        pl.BlockSpec((_BR, _W), functools.partial(
            lambda k, r, c: (r, k * _CB + c), k))
        for k in range(_K)
    ]
    return pl.pallas_call(
        _tc_rowsum_body,
        grid=grid,
        in_specs=stripe_specs,
        out_specs=pl.BlockSpec((_BR,), lambda r, c: (r,)),
        out_shape=jax.ShapeDtypeStruct((_B,), jnp.float32),
        scratch_shapes=[pltpu.VMEM((_BR, 128), jnp.float32)],
    )(*([x] * _K))


def _tc_combine_body(rs_ref, g_ref, t_ref, out_ref):
    rowsum = rs_ref[...]
    m = t_ref[...] != _PAD
    row_loss = jnp.where(m, _C1 - _CME * g_ref[...] - _EPS * rowsum, 0.0)
    out_ref[0, 0] = jnp.sum(row_loss)


def _tc_combine(rs_tc, g, t32):
    return pl.pallas_call(
        _tc_combine_body,
        out_specs=pl.BlockSpec(memory_space=pltpu.SMEM),
        out_shape=jax.ShapeDtypeStruct((1, 1), jnp.float32),
    )(rs_tc, g, t32)


def kernel(x, target):
    t32 = target.astype(jnp.int32)
    g = _sc_gather(x, t32)
    rs_tc = _tc_rowsum(x)
    loss = _tc_combine(rs_tc, g, t32)
    return loss[0, 0]
